# Initial kernel scaffold; baseline (speedup 1.0000x reference)
import jax
import jax.numpy as jnp
from jax.experimental import pallas as pl


def kernel(obs, g, Ws0, bs0, Wn0, bn0, Ws1, bs1, Wn1, bn1, M0, c0, M1, c1, M2, c2):
    raise NotImplementedError("write your pallas kernel here")


# speedup vs baseline: 5.2728x; 5.2728x over previous
import functools

import jax
import jax.numpy as jnp
from jax import lax
from jax.experimental import pallas as pl
from jax.experimental.pallas import tpu as pltpu
from jax.experimental.pallas import tpu_sc as plsc

_N = 32768
_E = 524288
_F = 128
_G = 64
_Q = 16
_C = 128
_NT = 16
_CHUNKS = _E // (_NT * _C)
_CHUNKS32 = _E // (2 * _NT * _C)
_RPT = _N // _NT

_sc_mesh = plsc.VectorSubcoreMesh(core_axis_name="c", subcore_axis_name="s")
_sc_params = pltpu.CompilerParams(use_tc_tiling_on_sc=False)


@functools.partial(
    pl.kernel,
    out_type=jax.ShapeDtypeStruct((2, _N, _Q), jnp.float32),
    mesh=_sc_mesh,
    scratch_types=[
        pltpu.VMEM((_CHUNKS32, _C), jnp.int32),
        pltpu.VMEM((_C, _Q), jnp.float32),
        pltpu.VMEM_SHARED((_N, _Q), jnp.float32),
    ],
    compiler_params=_sc_params,
)
def _sc_deg(dst_hbm, ones_hbm, zeros_hbm, out_hbm, dst_v, ones_v, acc):
    c = lax.axis_index("c")
    s = lax.axis_index("s")
    w = c * _NT + s
    pltpu.sync_copy(zeros_hbm.at[pl.ds(s * _RPT, _RPT)],
                    acc.at[pl.ds(s * _RPT, _RPT)])
    pltpu.sync_copy(dst_hbm.at[w], dst_v)
    pltpu.sync_copy(ones_hbm, ones_v)
    plsc.subcore_barrier()

    @pl.loop(0, _CHUNKS32)
    def _(j):
        pltpu.sync_copy(ones_v, acc.at[dst_v.at[j]], add=True)

    plsc.subcore_barrier()
    pltpu.sync_copy(acc.at[pl.ds(s * _RPT, _RPT)],
                    out_hbm.at[c, pl.ds(s * _RPT, _RPT)])


@functools.partial(
    pl.kernel,
    out_type=jax.ShapeDtypeStruct((4, _N, _Q), jnp.float32),
    mesh=_sc_mesh,
    scratch_types=[
        pltpu.VMEM((_CHUNKS, _C), jnp.int32),
        pltpu.VMEM((_CHUNKS, _C), jnp.int32),
        pltpu.VMEM((_C, _Q), jnp.float32),
        pltpu.VMEM_SHARED((_N, _Q), jnp.float32),
        pltpu.SemaphoreType.DMA,
    ],
    compiler_params=_sc_params,
)
def _sc_agg(p_hbm, src_hbm, dst_hbm, zeros_hbm, out_hbm,
            src_v, dst_v, rows_v, acc, sem):
    c = lax.axis_index("c")
    s = lax.axis_index("s")
    pltpu.sync_copy(src_hbm.at[s], src_v)
    pltpu.sync_copy(dst_hbm.at[s], dst_v)
    for p in range(2):
        q = c * 2 + p
        pltpu.sync_copy(zeros_hbm.at[pl.ds(s * _RPT, _RPT)],
                        acc.at[pl.ds(s * _RPT, _RPT)])
        plsc.subcore_barrier()

        @pl.loop(0, _CHUNKS)
        def _(j):
            pltpu.async_copy(p_hbm.at[q].at[src_v.at[j]], rows_v, sem).wait()
            pltpu.sync_copy(rows_v, acc.at[dst_v.at[j]], add=True)

        plsc.subcore_barrier()
        pltpu.sync_copy(acc.at[pl.ds(s * _RPT, _RPT)],
                        out_hbm.at[q, pl.ds(s * _RPT, _RPT)])
        plsc.subcore_barrier()


_BN = 2048


def _split_p(pn, p_ref):
    for q in range(4):
        p_ref[q] = pn[:, q * _Q:(q + 1) * _Q]


def _tc_pre_body(h_ref, ws_ref, bs_ref, wn_ref, s_ref, p_ref):
    hb = h_ref[...]
    s_ref[...] = jnp.maximum(
        jnp.dot(hb, ws_ref[...], preferred_element_type=jnp.float32, precision=jax.lax.Precision.HIGHEST)
        + bs_ref[...], 0.0)
    _split_p(jnp.dot(hb, wn_ref[...], preferred_element_type=jnp.float32, precision=jax.lax.Precision.HIGHEST),
             p_ref)


_tc_pre = pl.pallas_call(
    _tc_pre_body,
    grid=(_N // _BN,),
    in_specs=[
        pl.BlockSpec((_BN, _F), lambda i: (i, 0)),
        pl.BlockSpec((_F, _G), lambda i: (0, 0)),
        pl.BlockSpec((1, _G), lambda i: (0, 0)),
        pl.BlockSpec((_F, _G), lambda i: (0, 0)),
    ],
    out_specs=[
        pl.BlockSpec((_BN, _G), lambda i: (i, 0)),
        pl.BlockSpec((4, _BN, _Q), lambda i: (0, i, 0)),
    ],
    out_shape=[
        jax.ShapeDtypeStruct((_N, _G), jnp.float32),
        jax.ShapeDtypeStruct((4, _N, _Q), jnp.float32),
    ],
)


def _neigh(agg_ref, deg_ref, bn_ref):
    agg = jnp.concatenate([agg_ref[q] for q in range(4)], axis=1)
    deg = deg_ref[0][:, 0:1] + deg_ref[1][:, 0:1]
    inv = 1.0 / jnp.maximum(deg, 1.0)
    return jnp.maximum(agg * inv + bn_ref[...], 0.0)


def _tc_mid_body(agg_ref, deg_ref, s0_ref, bn0_ref, ws_ref, bs_ref, wn_ref,
                 s1_ref, p_ref):
    n0 = _neigh(agg_ref, deg_ref, bn0_ref)
    h1 = jnp.concatenate([s0_ref[...], n0], axis=1)
    s1_ref[...] = jnp.maximum(
        jnp.dot(h1, ws_ref[...], preferred_element_type=jnp.float32, precision=jax.lax.Precision.HIGHEST)
        + bs_ref[...], 0.0)
    _split_p(jnp.dot(h1, wn_ref[...], preferred_element_type=jnp.float32, precision=jax.lax.Precision.HIGHEST),
             p_ref)


_tc_mid = pl.pallas_call(
    _tc_mid_body,
    grid=(_N // _BN,),
    in_specs=[
        pl.BlockSpec((4, _BN, _Q), lambda i: (0, i, 0)),
        pl.BlockSpec((2, _BN, _Q), lambda i: (0, i, 0)),
        pl.BlockSpec((_BN, _G), lambda i: (i, 0)),
        pl.BlockSpec((1, _G), lambda i: (0, 0)),
        pl.BlockSpec((_F, _G), lambda i: (0, 0)),
        pl.BlockSpec((1, _G), lambda i: (0, 0)),
        pl.BlockSpec((_F, _G), lambda i: (0, 0)),
    ],
    out_specs=[
        pl.BlockSpec((_BN, _G), lambda i: (i, 0)),
        pl.BlockSpec((4, _BN, _Q), lambda i: (0, i, 0)),
    ],
    out_shape=[
        jax.ShapeDtypeStruct((_N, _G), jnp.float32),
        jax.ShapeDtypeStruct((4, _N, _Q), jnp.float32),
    ],
)


def _tc_post_body(agg_ref, deg_ref, s1_ref, bn1_ref, h2_ref):
    n1 = _neigh(agg_ref, deg_ref, bn1_ref)
    h2_ref[...] = jnp.concatenate([s1_ref[...], n1], axis=1)


_tc_post = pl.pallas_call(
    _tc_post_body,
    grid=(_N // _BN,),
    in_specs=[
        pl.BlockSpec((4, _BN, _Q), lambda i: (0, i, 0)),
        pl.BlockSpec((2, _BN, _Q), lambda i: (0, i, 0)),
        pl.BlockSpec((_BN, _G), lambda i: (i, 0)),
        pl.BlockSpec((1, _G), lambda i: (0, 0)),
    ],
    out_specs=pl.BlockSpec((_BN, _F), lambda i: (i, 0)),
    out_shape=jax.ShapeDtypeStruct((_N, _F), jnp.float32),
)

_BR = 512


def _tc_head_body(x_ref, m0_ref, c0_ref, m1_ref, c1_ref, m2_ref, c2_ref,
                  v_ref):
    t = jnp.tanh(
        jnp.dot(x_ref[...], m0_ref[...], preferred_element_type=jnp.float32, precision=jax.lax.Precision.HIGHEST)
        + c0_ref[...])
    t = jnp.tanh(
        jnp.dot(t, m1_ref[...], preferred_element_type=jnp.float32, precision=jax.lax.Precision.HIGHEST)
        + c1_ref[...])
    v_ref[...] = jnp.sum(t * m2_ref[...], axis=1, keepdims=True) + c2_ref[...]


_tc_head = pl.pallas_call(
    _tc_head_body,
    grid=(_N // 4 // _BR,),
    in_specs=[
        pl.BlockSpec((_BR, 512), lambda i: (i, 0)),
        pl.BlockSpec((512, 256), lambda i: (0, 0)),
        pl.BlockSpec((1, 256), lambda i: (0, 0)),
        pl.BlockSpec((256, 256), lambda i: (0, 0)),
        pl.BlockSpec((1, 256), lambda i: (0, 0)),
        pl.BlockSpec((1, 256), lambda i: (0, 0)),
        pl.BlockSpec((1, 1), lambda i: (0, 0)),
    ],
    out_specs=pl.BlockSpec((_BR, 1), lambda i: (i, 0)),
    out_shape=jax.ShapeDtypeStruct((_N // 4, 1), jnp.float32),
)


def kernel(obs, g, Ws0, bs0, Wn0, bn0, Ws1, bs1, Wn1, bn1,
           M0, c0, M1, c1, M2, c2):
    steps = obs.shape[0]
    h = obs.reshape(_N, _F)
    src = g[0].astype(jnp.int32).reshape(_NT, _CHUNKS, _C)
    dst = g[1].astype(jnp.int32)
    dst16 = dst.reshape(_NT, _CHUNKS, _C)
    dst32 = dst.reshape(2 * _NT, _CHUNKS32, _C)
    zeros_q = jnp.zeros((_N, _Q), jnp.float32)
    ones = jnp.ones((_C, _Q), jnp.float32)

    degs = _sc_deg(dst32, ones, zeros_q)
    s0, p0 = _tc_pre(h, Ws0, bs0.reshape(1, -1), Wn0)
    agg0 = _sc_agg(p0, src, dst16, zeros_q)
    s1, p1 = _tc_mid(agg0, degs, s0, bn0.reshape(1, -1),
                     Ws1, bs1.reshape(1, -1), Wn1)
    agg1 = _sc_agg(p1, src, dst16, zeros_q)
    h2 = _tc_post(agg1, degs, s1, bn1.reshape(1, -1))
    x = h2.reshape(_N // 4, 512)
    v = _tc_head(x, M0, c0.reshape(1, -1), M1, c1.reshape(1, -1),
                 M2.reshape(1, -1), c2.reshape(1, 1))
    return v.reshape(steps, 8, 16)


# speedup vs baseline: 9.7293x; 1.8452x over previous
import functools

import jax
import jax.numpy as jnp
from jax import lax
from jax.experimental import pallas as pl
from jax.experimental.pallas import tpu as pltpu
from jax.experimental.pallas import tpu_sc as plsc

_N = 32768
_E = 524288
_F = 128
_G = 64
_Q = 16
_C = 128
_NT = 16
_CHUNKS = _E // (_NT * _C)
_CHUNKS32 = _E // (2 * _NT * _C)
_RPT = _N // _NT

_sc_mesh = plsc.VectorSubcoreMesh(core_axis_name="c", subcore_axis_name="s")
_sc_params = pltpu.CompilerParams(use_tc_tiling_on_sc=False)


@functools.partial(
    pl.kernel,
    out_type=jax.ShapeDtypeStruct((2, _N, _Q), jnp.float32),
    mesh=_sc_mesh,
    scratch_types=[
        pltpu.VMEM((_CHUNKS32, _C), jnp.int32),
        pltpu.VMEM((_C, _Q), jnp.float32),
        pltpu.VMEM_SHARED((_N, _Q), jnp.float32),
    ],
    compiler_params=_sc_params,
)
def _sc_deg(dst_hbm, ones_hbm, zeros_hbm, out_hbm, dst_v, ones_v, acc):
    c = lax.axis_index("c")
    s = lax.axis_index("s")
    w = c * _NT + s
    pltpu.sync_copy(zeros_hbm.at[pl.ds(s * _RPT, _RPT)],
                    acc.at[pl.ds(s * _RPT, _RPT)])
    pltpu.sync_copy(dst_hbm.at[w], dst_v)
    pltpu.sync_copy(ones_hbm, ones_v)
    plsc.subcore_barrier()

    @pl.loop(0, _CHUNKS32)
    def _(j):
        pltpu.sync_copy(ones_v, acc.at[dst_v.at[j]], add=True)

    plsc.subcore_barrier()
    pltpu.sync_copy(acc.at[pl.ds(s * _RPT, _RPT)],
                    out_hbm.at[c, pl.ds(s * _RPT, _RPT)])


_NB = 4


@functools.partial(
    pl.kernel,
    out_type=jax.ShapeDtypeStruct((4, _N, _Q), jnp.float32),
    mesh=_sc_mesh,
    scratch_types=[
        pltpu.VMEM((_CHUNKS, _C), jnp.int32),
        pltpu.VMEM((_CHUNKS, _C), jnp.int32),
        [pltpu.VMEM((_C, _Q), jnp.float32) for _ in range(_NB)],
        pltpu.VMEM_SHARED((_N, _Q), jnp.float32),
        [pltpu.SemaphoreType.DMA for _ in range(_NB)],
        [pltpu.SemaphoreType.DMA for _ in range(_NB)],
    ],
    compiler_params=_sc_params,
)
def _sc_agg(p_hbm, src_hbm, dst_hbm, zeros_hbm, out_hbm,
            src_v, dst_v, rows, acc, gsem, ssem):
    c = lax.axis_index("c")
    s = lax.axis_index("s")
    pltpu.sync_copy(src_hbm.at[s], src_v)
    pltpu.sync_copy(dst_hbm.at[s], dst_v)
    for p in range(2):
        q = c * 2 + p
        pltpu.sync_copy(zeros_hbm.at[pl.ds(s * _RPT, _RPT)],
                        acc.at[pl.ds(s * _RPT, _RPT)])
        plsc.subcore_barrier()

        for b in range(_NB):
            pltpu.async_copy(p_hbm.at[q].at[src_v.at[b]], rows[b], gsem[b])

        @pl.loop(0, _CHUNKS // _NB)
        def _(i):
            j0 = i * _NB
            for b in range(_NB):
                pltpu.make_async_copy(
                    p_hbm.at[q].at[src_v.at[j0 + b]], rows[b],
                    gsem[b]).wait()
                pltpu.async_copy(rows[b], acc.at[dst_v.at[j0 + b]], ssem[b],
                                 add=True)
            for b in range(_NB):
                pltpu.make_async_copy(
                    rows[b], acc.at[dst_v.at[j0 + b]], ssem[b]).wait()
                nj = j0 + _NB + b

                @pl.when(nj < _CHUNKS)
                def _():
                    pltpu.async_copy(p_hbm.at[q].at[src_v.at[nj]], rows[b],
                                     gsem[b])

        plsc.subcore_barrier()
        pltpu.sync_copy(acc.at[pl.ds(s * _RPT, _RPT)],
                        out_hbm.at[q, pl.ds(s * _RPT, _RPT)])
        plsc.subcore_barrier()


_BN = 2048


def _split_p(pn, p_ref):
    for q in range(4):
        p_ref[q] = pn[:, q * _Q:(q + 1) * _Q]


def _tc_pre_body(h_ref, ws_ref, bs_ref, wn_ref, s_ref, p_ref):
    hb = h_ref[...]
    s_ref[...] = jnp.maximum(
        jnp.dot(hb, ws_ref[...], preferred_element_type=jnp.float32, precision=jax.lax.Precision.HIGHEST)
        + bs_ref[...], 0.0)
    _split_p(jnp.dot(hb, wn_ref[...], preferred_element_type=jnp.float32, precision=jax.lax.Precision.HIGHEST),
             p_ref)


_tc_pre = pl.pallas_call(
    _tc_pre_body,
    grid=(_N // _BN,),
    in_specs=[
        pl.BlockSpec((_BN, _F), lambda i: (i, 0)),
        pl.BlockSpec((_F, _G), lambda i: (0, 0)),
        pl.BlockSpec((1, _G), lambda i: (0, 0)),
        pl.BlockSpec((_F, _G), lambda i: (0, 0)),
    ],
    out_specs=[
        pl.BlockSpec((_BN, _G), lambda i: (i, 0)),
        pl.BlockSpec((4, _BN, _Q), lambda i: (0, i, 0)),
    ],
    out_shape=[
        jax.ShapeDtypeStruct((_N, _G), jnp.float32),
        jax.ShapeDtypeStruct((4, _N, _Q), jnp.float32),
    ],
)


def _neigh(agg_ref, deg_ref, bn_ref):
    agg = jnp.concatenate([agg_ref[q] for q in range(4)], axis=1)
    deg = deg_ref[0][:, 0:1] + deg_ref[1][:, 0:1]
    inv = 1.0 / jnp.maximum(deg, 1.0)
    return jnp.maximum(agg * inv + bn_ref[...], 0.0)


def _tc_mid_body(agg_ref, deg_ref, s0_ref, bn0_ref, ws_ref, bs_ref, wn_ref,
                 s1_ref, p_ref):
    n0 = _neigh(agg_ref, deg_ref, bn0_ref)
    h1 = jnp.concatenate([s0_ref[...], n0], axis=1)
    s1_ref[...] = jnp.maximum(
        jnp.dot(h1, ws_ref[...], preferred_element_type=jnp.float32, precision=jax.lax.Precision.HIGHEST)
        + bs_ref[...], 0.0)
    _split_p(jnp.dot(h1, wn_ref[...], preferred_element_type=jnp.float32, precision=jax.lax.Precision.HIGHEST),
             p_ref)


_tc_mid = pl.pallas_call(
    _tc_mid_body,
    grid=(_N // _BN,),
    in_specs=[
        pl.BlockSpec((4, _BN, _Q), lambda i: (0, i, 0)),
        pl.BlockSpec((2, _BN, _Q), lambda i: (0, i, 0)),
        pl.BlockSpec((_BN, _G), lambda i: (i, 0)),
        pl.BlockSpec((1, _G), lambda i: (0, 0)),
        pl.BlockSpec((_F, _G), lambda i: (0, 0)),
        pl.BlockSpec((1, _G), lambda i: (0, 0)),
        pl.BlockSpec((_F, _G), lambda i: (0, 0)),
    ],
    out_specs=[
        pl.BlockSpec((_BN, _G), lambda i: (i, 0)),
        pl.BlockSpec((4, _BN, _Q), lambda i: (0, i, 0)),
    ],
    out_shape=[
        jax.ShapeDtypeStruct((_N, _G), jnp.float32),
        jax.ShapeDtypeStruct((4, _N, _Q), jnp.float32),
    ],
)


def _tc_post_body(agg_ref, deg_ref, s1_ref, bn1_ref, h2_ref):
    n1 = _neigh(agg_ref, deg_ref, bn1_ref)
    h2_ref[...] = jnp.concatenate([s1_ref[...], n1], axis=1)


_tc_post = pl.pallas_call(
    _tc_post_body,
    grid=(_N // _BN,),
    in_specs=[
        pl.BlockSpec((4, _BN, _Q), lambda i: (0, i, 0)),
        pl.BlockSpec((2, _BN, _Q), lambda i: (0, i, 0)),
        pl.BlockSpec((_BN, _G), lambda i: (i, 0)),
        pl.BlockSpec((1, _G), lambda i: (0, 0)),
    ],
    out_specs=pl.BlockSpec((_BN, _F), lambda i: (i, 0)),
    out_shape=jax.ShapeDtypeStruct((_N, _F), jnp.float32),
)

_BR = 512


def _tc_head_body(x_ref, m0_ref, c0_ref, m1_ref, c1_ref, m2_ref, c2_ref,
                  v_ref):
    t = jnp.tanh(
        jnp.dot(x_ref[...], m0_ref[...], preferred_element_type=jnp.float32, precision=jax.lax.Precision.HIGHEST)
        + c0_ref[...])
    t = jnp.tanh(
        jnp.dot(t, m1_ref[...], preferred_element_type=jnp.float32, precision=jax.lax.Precision.HIGHEST)
        + c1_ref[...])
    v_ref[...] = jnp.sum(t * m2_ref[...], axis=1, keepdims=True) + c2_ref[...]


_tc_head = pl.pallas_call(
    _tc_head_body,
    grid=(_N // 4 // _BR,),
    in_specs=[
        pl.BlockSpec((_BR, 512), lambda i: (i, 0)),
        pl.BlockSpec((512, 256), lambda i: (0, 0)),
        pl.BlockSpec((1, 256), lambda i: (0, 0)),
        pl.BlockSpec((256, 256), lambda i: (0, 0)),
        pl.BlockSpec((1, 256), lambda i: (0, 0)),
        pl.BlockSpec((1, 256), lambda i: (0, 0)),
        pl.BlockSpec((1, 1), lambda i: (0, 0)),
    ],
    out_specs=pl.BlockSpec((_BR, 1), lambda i: (i, 0)),
    out_shape=jax.ShapeDtypeStruct((_N // 4, 1), jnp.float32),
)


def kernel(obs, g, Ws0, bs0, Wn0, bn0, Ws1, bs1, Wn1, bn1,
           M0, c0, M1, c1, M2, c2):
    steps = obs.shape[0]
    h = obs.reshape(_N, _F)
    src = g[0].astype(jnp.int32).reshape(_NT, _CHUNKS, _C)
    dst = g[1].astype(jnp.int32)
    dst16 = dst.reshape(_NT, _CHUNKS, _C)
    dst32 = dst.reshape(2 * _NT, _CHUNKS32, _C)
    zeros_q = jnp.zeros((_N, _Q), jnp.float32)
    ones = jnp.ones((_C, _Q), jnp.float32)

    degs = _sc_deg(dst32, ones, zeros_q)
    s0, p0 = _tc_pre(h, Ws0, bs0.reshape(1, -1), Wn0)
    agg0 = _sc_agg(p0, src, dst16, zeros_q)
    s1, p1 = _tc_mid(agg0, degs, s0, bn0.reshape(1, -1),
                     Ws1, bs1.reshape(1, -1), Wn1)
    agg1 = _sc_agg(p1, src, dst16, zeros_q)
    h2 = _tc_post(agg1, degs, s1, bn1.reshape(1, -1))
    x = h2.reshape(_N // 4, 512)
    v = _tc_head(x, M0, c0.reshape(1, -1), M1, c1.reshape(1, -1),
                 M2.reshape(1, -1), c2.reshape(1, 1))
    return v.reshape(steps, 8, 16)


# speedup vs baseline: 12.2833x; 1.2625x over previous
import functools

import jax
import jax.numpy as jnp
from jax import lax
from jax.experimental import pallas as pl
from jax.experimental.pallas import tpu as pltpu
from jax.experimental.pallas import tpu_sc as plsc

_N = 32768
_E = 524288
_F = 128
_G = 64
_Q = 16
_C = 128
_NT = 16
_CHUNKS = _E // (_NT * _C)
_CHUNKS32 = _E // (2 * _NT * _C)
_RPT = _N // _NT

_HIGH = jax.lax.Precision.HIGHEST

_sc_mesh = plsc.VectorSubcoreMesh(core_axis_name="c", subcore_axis_name="s")
_sc_params = pltpu.CompilerParams(use_tc_tiling_on_sc=False)


@functools.partial(
    pl.kernel,
    out_type=jax.ShapeDtypeStruct((2, _N, _Q), jnp.float32),
    mesh=_sc_mesh,
    scratch_types=[
        pltpu.VMEM((_CHUNKS32, _C), jnp.int32),
        pltpu.VMEM((_C, _Q), jnp.float32),
        pltpu.VMEM_SHARED((_N, _Q), jnp.float32),
    ],
    compiler_params=_sc_params,
)
def _sc_deg(dst_hbm, ones_hbm, zeros_hbm, out_hbm, dst_v, ones_v, acc):
    c = lax.axis_index("c")
    s = lax.axis_index("s")
    w = c * _NT + s
    pltpu.sync_copy(zeros_hbm.at[pl.ds(s * _RPT, _RPT)],
                    acc.at[pl.ds(s * _RPT, _RPT)])
    pltpu.sync_copy(dst_hbm.at[w], dst_v)
    pltpu.sync_copy(ones_hbm, ones_v)
    plsc.subcore_barrier()

    @pl.loop(0, _CHUNKS32)
    def _(j):
        pltpu.sync_copy(ones_v, acc.at[dst_v.at[j]], add=True)

    plsc.subcore_barrier()
    pltpu.sync_copy(acc.at[pl.ds(s * _RPT, _RPT)],
                    out_hbm.at[c, pl.ds(s * _RPT, _RPT)])


_NB = 8


@functools.partial(
    pl.kernel,
    out_type=jax.ShapeDtypeStruct((_N, _G), jnp.float32),
    mesh=_sc_mesh,
    scratch_types=[
        pltpu.VMEM((_CHUNKS, _C), jnp.int32),
        pltpu.VMEM((_CHUNKS, _C), jnp.int32),
        [pltpu.VMEM((_C, _Q), jnp.float32) for _ in range(_NB)],
        pltpu.VMEM_SHARED((_N, _Q), jnp.float32),
        [pltpu.SemaphoreType.DMA for _ in range(_NB)],
        [pltpu.SemaphoreType.DMA for _ in range(_NB)],
    ],
    compiler_params=_sc_params,
)
def _sc_agg(p_hbm, src_hbm, dst_hbm, zeros_hbm, out_hbm,
            src_v, dst_v, rows, acc, gsem, ssem):
    c = lax.axis_index("c")
    s = lax.axis_index("s")
    pltpu.sync_copy(src_hbm.at[s], src_v)
    pltpu.sync_copy(dst_hbm.at[s], dst_v)
    for p in range(2):
        q = c * 2 + p
        pltpu.sync_copy(zeros_hbm.at[pl.ds(s * _RPT, _RPT)],
                        acc.at[pl.ds(s * _RPT, _RPT)])
        plsc.subcore_barrier()

        for b in range(_NB):
            pltpu.async_copy(p_hbm.at[q].at[src_v.at[b]], rows[b], gsem[b])

        @pl.loop(0, _CHUNKS // _NB)
        def _(i):
            j0 = i * _NB
            for b in range(_NB):
                pltpu.make_async_copy(
                    p_hbm.at[q].at[src_v.at[j0 + b]], rows[b],
                    gsem[b]).wait()
                pltpu.async_copy(rows[b], acc.at[dst_v.at[j0 + b]], ssem[b],
                                 add=True)
            for b in range(_NB):
                pltpu.make_async_copy(
                    rows[b], acc.at[dst_v.at[j0 + b]], ssem[b]).wait()
                nj = j0 + _NB + b

                @pl.when(nj < _CHUNKS)
                def _():
                    pltpu.async_copy(p_hbm.at[q].at[src_v.at[nj]], rows[b],
                                     gsem[b])

        plsc.subcore_barrier()
        pltpu.sync_copy(acc.at[pl.ds(s * _RPT, _RPT)],
                        out_hbm.at[pl.ds(s * _RPT, _RPT), pl.ds(q * _Q, _Q)])
        plsc.subcore_barrier()


_BN = 2048


def _split_p(pn, p_ref):
    for q in range(4):
        p_ref[q] = pn[:, q * _Q:(q + 1) * _Q]


def _tc_pre_body(h_ref, ws_ref, bs_ref, wn_ref, s_ref, p_ref):
    hb = h_ref[...]
    s_ref[...] = jnp.maximum(
        jnp.dot(hb, ws_ref[...], preferred_element_type=jnp.float32,
                precision=_HIGH) + bs_ref[...], 0.0)
    _split_p(jnp.dot(hb, wn_ref[...], preferred_element_type=jnp.float32,
                     precision=_HIGH), p_ref)


_tc_pre = pl.pallas_call(
    _tc_pre_body,
    grid=(_N // _BN,),
    in_specs=[
        pl.BlockSpec((_BN, _F), lambda i: (i, 0)),
        pl.BlockSpec((_F, _G), lambda i: (0, 0)),
        pl.BlockSpec((1, _G), lambda i: (0, 0)),
        pl.BlockSpec((_F, _G), lambda i: (0, 0)),
    ],
    out_specs=[
        pl.BlockSpec((_BN, _G), lambda i: (i, 0)),
        pl.BlockSpec((4, _BN, _Q), lambda i: (0, i, 0)),
    ],
    out_shape=[
        jax.ShapeDtypeStruct((_N, _G), jnp.float32),
        jax.ShapeDtypeStruct((4, _N, _Q), jnp.float32),
    ],
)


def _tc_mid_body(agg_ref, deg_ref, s0_ref, bn_ref, ws_ref, bs_ref, wn_ref,
                 s1_ref, p_ref):
    deg = deg_ref[0][:, 0:1] + deg_ref[1][:, 0:1]
    inv = 1.0 / jnp.maximum(deg, 1.0)
    n0 = jnp.maximum(agg_ref[...] * inv + bn_ref[...], 0.0)
    h1 = jnp.concatenate([s0_ref[...], n0], axis=1)
    s1_ref[...] = jnp.maximum(
        jnp.dot(h1, ws_ref[...], preferred_element_type=jnp.float32,
                precision=_HIGH) + bs_ref[...], 0.0)
    _split_p(jnp.dot(h1, wn_ref[...], preferred_element_type=jnp.float32,
                     precision=_HIGH), p_ref)


_tc_mid = pl.pallas_call(
    _tc_mid_body,
    grid=(_N // _BN,),
    in_specs=[
        pl.BlockSpec((_BN, _G), lambda i: (i, 0)),
        pl.BlockSpec((2, _BN, _Q), lambda i: (0, i, 0)),
        pl.BlockSpec((_BN, _G), lambda i: (i, 0)),
        pl.BlockSpec((1, _G), lambda i: (0, 0)),
        pl.BlockSpec((_F, _G), lambda i: (0, 0)),
        pl.BlockSpec((1, _G), lambda i: (0, 0)),
        pl.BlockSpec((_F, _G), lambda i: (0, 0)),
    ],
    out_specs=[
        pl.BlockSpec((_BN, _G), lambda i: (i, 0)),
        pl.BlockSpec((4, _BN, _Q), lambda i: (0, i, 0)),
    ],
    out_shape=[
        jax.ShapeDtypeStruct((_N, _G), jnp.float32),
        jax.ShapeDtypeStruct((4, _N, _Q), jnp.float32),
    ],
)

_BR = 1024


def _tc_head_body(sg_ref, ag_ref, ig_ref, bn_ref, m0s_ref, m0n_ref, c0_ref,
                  m1_ref, c1_ref, m2_ref, c2_ref, v_ref):
    n1 = jnp.maximum(ag_ref[...] * ig_ref[...] + bn_ref[...], 0.0)
    t = jnp.tanh(
        jnp.dot(sg_ref[...], m0s_ref[...], preferred_element_type=jnp.float32,
                precision=_HIGH)
        + jnp.dot(n1, m0n_ref[...], preferred_element_type=jnp.float32,
                  precision=_HIGH)
        + c0_ref[...])
    t = jnp.tanh(
        jnp.dot(t, m1_ref[...], preferred_element_type=jnp.float32,
                precision=_HIGH) + c1_ref[...])
    v_ref[...] = jnp.sum(t * m2_ref[...], axis=1, keepdims=True) + c2_ref[...]


_tc_head = pl.pallas_call(
    _tc_head_body,
    grid=(_N // 4 // _BR,),
    in_specs=[
        pl.BlockSpec((_BR, 256), lambda i: (i, 0)),
        pl.BlockSpec((_BR, 256), lambda i: (i, 0)),
        pl.BlockSpec((_BR, 256), lambda i: (i, 0)),
        pl.BlockSpec((1, 256), lambda i: (0, 0)),
        pl.BlockSpec((256, 256), lambda i: (0, 0)),
        pl.BlockSpec((256, 256), lambda i: (0, 0)),
        pl.BlockSpec((1, 256), lambda i: (0, 0)),
        pl.BlockSpec((256, 256), lambda i: (0, 0)),
        pl.BlockSpec((1, 256), lambda i: (0, 0)),
        pl.BlockSpec((1, 256), lambda i: (0, 0)),
        pl.BlockSpec((1, 1), lambda i: (0, 0)),
    ],
    out_specs=pl.BlockSpec((_BR, 1), lambda i: (i, 0)),
    out_shape=jax.ShapeDtypeStruct((_N // 4, 1), jnp.float32),
)


def kernel(obs, g, Ws0, bs0, Wn0, bn0, Ws1, bs1, Wn1, bn1,
           M0, c0, M1, c1, M2, c2):
    steps = obs.shape[0]
    h = obs.reshape(_N, _F)
    src = g[0].astype(jnp.int32).reshape(_NT, _CHUNKS, _C)
    dst = g[1].astype(jnp.int32)
    dst16 = dst.reshape(_NT, _CHUNKS, _C)
    dst32 = dst.reshape(2 * _NT, _CHUNKS32, _C)
    zeros_q = jnp.zeros((_N, _Q), jnp.float32)
    ones = jnp.ones((_C, _Q), jnp.float32)

    m0r = M0.reshape(4, 2 * _G, 256)
    M0s = m0r[:, :_G, :].reshape(4 * _G, 256)
    M0n = m0r[:, _G:, :].reshape(4 * _G, 256)
    bn1_t = jnp.tile(bn1, 4).reshape(1, 4 * _G)

    degs = _sc_deg(dst32, ones, zeros_q)
    inv = 1.0 / jnp.maximum(degs[0, :, 0:1] + degs[1, :, 0:1], 1.0)
    inv_g = jnp.broadcast_to(inv, (_N, _G)).reshape(_N // 4, 4 * _G)

    s0, p0 = _tc_pre(h, Ws0, bs0.reshape(1, -1), Wn0)
    agg0 = _sc_agg(p0, src, dst16, zeros_q)
    s1, p1 = _tc_mid(agg0, degs, s0, bn0.reshape(1, -1),
                     Ws1, bs1.reshape(1, -1), Wn1)
    agg1 = _sc_agg(p1, src, dst16, zeros_q)
    v = _tc_head(s1.reshape(_N // 4, 4 * _G), agg1.reshape(_N // 4, 4 * _G),
                 inv_g, bn1_t, M0s, M0n, c0.reshape(1, -1),
                 M1, c1.reshape(1, -1), M2.reshape(1, -1),
                 c2.reshape(1, 1))
    return v.reshape(steps, 8, 16)
